# baseline (speedup 1.0000x reference)
import functools

import numpy as np
import jax
import jax.numpy as jnp
from jax import lax
from jax.experimental import pallas as pl
from jax.experimental.pallas import tpu as pltpu
from jax.experimental.pallas import tpu_sc as plsc

_TOPK_RATIO = 0.3
_TEMPERATURE = 10.0
_MIN_WEIGHT = 0.1
_SPATIAL_SCALE = 2.0


def _upsample_1d_matrix(n_in: int, n_out: int) -> np.ndarray:
    U = np.zeros((n_out, n_in), np.float64)
    for i in range(n_out):
        src = (i + 0.5) * (n_in / n_out) - 0.5
        j0 = int(np.floor(src))
        f = src - j0
        j0c = min(max(j0, 0), n_in - 1)
        j1c = min(max(j0 + 1, 0), n_in - 1)
        U[i, j0c] += 1.0 - f
        U[i, j1c] += f
    return U


@functools.lru_cache(maxsize=None)
def _upsample_2d_matrix(h_in: int, w_in: int, h_out: int, w_out: int):
    UH = _upsample_1d_matrix(h_in, h_out)
    UW = _upsample_1d_matrix(w_in, w_out)
    M = np.einsum("hk,wl->klhw", UH, UW).reshape(h_in * w_in, h_out * w_out)
    return np.asarray(M, np.float32)


def _mask_kernel(rank, a_ref, m_ref, mask_ref):
    a = a_ref[...]
    u = jnp.dot(a, m_ref[...], preferred_element_type=jnp.float32)
    ibits = jax.lax.bitcast_convert_type(u, jnp.int32)
    key = ibits ^ ((ibits >> 31) & jnp.int32(0x7FFFFFFF))

    B = a.shape[0]

    def body(b, t):
        t_try = t + (jnp.int32(1) << (jnp.int32(31) - b))
        cnt = jnp.sum((key < t_try).astype(jnp.int32), axis=1, keepdims=True)
        return jnp.where(cnt <= rank, t_try, t)

    t0 = jnp.full((B, 1), jnp.int32(-2147483648))
    t = jax.lax.fori_loop(0, 32, body, t0)
    thr_i = t ^ ((t >> 31) & jnp.int32(0x7FFFFFFF))
    thr = jax.lax.bitcast_convert_type(thr_i, jnp.float32)
    mask_ref[...] = jax.nn.sigmoid(_TEMPERATURE * (u - thr))


_BISECT_ITERS = 30


def _sc_mask_kernel(rank, B, al_hbm, ar_hbm, out_hbm, al_v, ar_v, aw_v, u_v, m_v):
    nc = 2
    wid = lax.axis_index("s") * nc + lax.axis_index("c")

    @pl.when(wid < B)
    def _():
        pltpu.sync_copy(al_hbm.at[wid], al_v)
        pltpu.sync_copy(ar_hbm.at[wid], ar_v)

        iota = lax.iota(jnp.int32, 16)
        w0s = [
            0.25 + 0.5 * (((iota + 16 * c) & 1).astype(jnp.float32))
            for c in range(3)
        ]

        def wpass(k, _):
            for c in range(3):
                sl = pl.ds(k * 48 + 16 * c, 16)
                aw_v[sl] = w0s[c] * al_v[sl] + (1.0 - w0s[c]) * ar_v[sl]
            return 0

        lax.fori_loop(0, 24, wpass, 0)

        for h in range(48):
            j = h // 2
            if h % 2 == 0:
                k0, wh0, k1, wh1 = max(j - 1, 0), 0.25, j, 0.75
            else:
                k0, wh0, k1, wh1 = j, 0.75, min(j + 1, 23), 0.25
            for c in range(3):
                u_v[pl.ds(h * 48 + 16 * c, 16)] = (
                    wh0 * aw_v[pl.ds(k0 * 48 + 16 * c, 16)]
                    + wh1 * aw_v[pl.ds(k1 * 48 + 16 * c, 16)]
                )

        def mmbody(i, c):
            mn, mx = c
            uc = u_v[pl.ds(i * 16, 16)]
            return jnp.minimum(mn, uc), jnp.maximum(mx, uc)

        mnv, mxv = lax.fori_loop(
            0, 144, mmbody, (u_v[pl.ds(0, 16)], u_v[pl.ds(0, 16)])
        )
        lo = mnv[0]
        hi = mxv[0]
        for j in range(1, 16):
            lo = jnp.minimum(lo, mnv[j])
            hi = jnp.maximum(hi, mxv[j])

        rankf = jnp.float32(rank)

        def bis(_, c):
            lo, hi = c
            mid = 0.5 * (lo + hi)

            def cnt(i, acc):
                uc = u_v[pl.ds(i * 16, 16)]
                return acc + jnp.where(uc < mid, 1.0, 0.0)

            acc = lax.fori_loop(0, 144, cnt, jnp.zeros((16,), jnp.float32))
            s = acc[0]
            for j in range(1, 16):
                s = s + acc[j]
            ok = s <= rankf
            return jnp.where(ok, mid, lo), jnp.where(ok, hi, mid)

        lo, hi = lax.fori_loop(0, _BISECT_ITERS, bis, (lo, hi))
        thr = lo

        def sbody(i, _):
            uc = u_v[pl.ds(i * 16, 16)]
            m_v[pl.ds(i * 16, 16)] = 1.0 / (
                1.0 + jnp.exp(-_TEMPERATURE * (uc - thr))
            )
            return 0

        lax.fori_loop(0, 144, sbody, 0)
        pltpu.sync_copy(m_v, out_hbm.at[wid])


def _sc_mask(al, ar, rank):
    B, _ = al.shape
    mesh = plsc.VectorSubcoreMesh(core_axis_name="c", subcore_axis_name="s")
    return pl.kernel(
        functools.partial(_sc_mask_kernel, rank, B),
        out_type=jax.ShapeDtypeStruct((B, 2304), jnp.float32),
        mesh=mesh,
        scratch_types=[
            pltpu.VMEM((1152,), jnp.float32),
            pltpu.VMEM((1152,), jnp.float32),
            pltpu.VMEM((1152,), jnp.float32),
            pltpu.VMEM((2304,), jnp.float32),
            pltpu.VMEM((2304,), jnp.float32),
        ],
    )(al, ar)


def _mul_kernel(H, mt_ref, f_ref, o_ref):
    for h in range(H):
        col = mt_ref[0, :, h : h + 1] + _MIN_WEIGHT
        o_ref[0, h] = f_ref[0, h] * col


def kernel(local_feat, attention_map):
    B, C, H, W = local_feat.shape
    Bb, C1, Hg, Wg = attention_map.shape
    Hu = int(Hg * _SPATIAL_SCALE)
    Wu = int(Wg * _SPATIAL_SCALE)
    assert (Hu, Wu) == (H, W) and C1 == 1 and Bb == B
    N = Hu * Wu
    k = int(_TOPK_RATIO * N)
    rank = N - k - 1

    wv = np.arange(Wu)
    l0 = np.maximum((wv - 1) >> 1, 0)
    l1 = np.minimum((wv + 1) >> 1, Wg - 1)
    att3 = attention_map.reshape(B, Hg, Wg)
    al = att3[:, :, l0].reshape(B, Hg * Wu)
    ar = att3[:, :, l1].reshape(B, Hg * Wu)

    mask_flat = _sc_mask(al, ar, rank)
    mask = mask_flat.reshape(B, 1, Hu, Wu)
    mask_t = mask_flat.reshape(B, Hu, Wu).transpose(0, 2, 1)

    feat_t = jnp.transpose(local_feat, (0, 2, 3, 1))
    weighted_t = pl.pallas_call(
        functools.partial(_mul_kernel, H),
        grid=(B,),
        in_specs=[
            pl.BlockSpec((1, W, H), lambda b: (b, 0, 0)),
            pl.BlockSpec((1, H, W, C), lambda b: (b, 0, 0, 0)),
        ],
        out_specs=pl.BlockSpec((1, H, W, C), lambda b: (b, 0, 0, 0)),
        out_shape=jax.ShapeDtypeStruct((B, H, W, C), jnp.float32),
    )(mask_t, feat_t)
    weighted = jnp.transpose(weighted_t, (0, 3, 1, 2))

    return weighted, mask


# speedup vs baseline: 1.1694x; 1.1694x over previous
import functools

import numpy as np
import jax
import jax.numpy as jnp
from jax import lax
from jax.experimental import pallas as pl
from jax.experimental.pallas import tpu as pltpu
from jax.experimental.pallas import tpu_sc as plsc

_TOPK_RATIO = 0.3
_TEMPERATURE = 10.0
_MIN_WEIGHT = 0.1
_SPATIAL_SCALE = 2.0


def _upsample_1d_matrix(n_in: int, n_out: int) -> np.ndarray:
    U = np.zeros((n_out, n_in), np.float64)
    for i in range(n_out):
        src = (i + 0.5) * (n_in / n_out) - 0.5
        j0 = int(np.floor(src))
        f = src - j0
        j0c = min(max(j0, 0), n_in - 1)
        j1c = min(max(j0 + 1, 0), n_in - 1)
        U[i, j0c] += 1.0 - f
        U[i, j1c] += f
    return U


@functools.lru_cache(maxsize=None)
def _upsample_2d_matrix(h_in: int, w_in: int, h_out: int, w_out: int):
    UH = _upsample_1d_matrix(h_in, h_out)
    UW = _upsample_1d_matrix(w_in, w_out)
    M = np.einsum("hk,wl->klhw", UH, UW).reshape(h_in * w_in, h_out * w_out)
    return np.asarray(M, np.float32)


def _mask_kernel(rank, a_ref, m_ref, mask_ref):
    a = a_ref[...]
    u = jnp.dot(a, m_ref[...], preferred_element_type=jnp.float32)
    ibits = jax.lax.bitcast_convert_type(u, jnp.int32)
    key = ibits ^ ((ibits >> 31) & jnp.int32(0x7FFFFFFF))

    B = a.shape[0]

    def body(b, t):
        t_try = t + (jnp.int32(1) << (jnp.int32(31) - b))
        cnt = jnp.sum((key < t_try).astype(jnp.int32), axis=1, keepdims=True)
        return jnp.where(cnt <= rank, t_try, t)

    t0 = jnp.full((B, 1), jnp.int32(-2147483648))
    t = jax.lax.fori_loop(0, 32, body, t0)
    thr_i = t ^ ((t >> 31) & jnp.int32(0x7FFFFFFF))
    thr = jax.lax.bitcast_convert_type(thr_i, jnp.float32)
    mask_ref[...] = jax.nn.sigmoid(_TEMPERATURE * (u - thr))


_BISECT_ITERS = 22


def _sc_mask_kernel(rank, B, al_hbm, ar_hbm, out_hbm, al_v, ar_v, aw_v, u_v, m_v):
    nc = 2
    wid = lax.axis_index("s") * nc + lax.axis_index("c")

    @pl.when(wid < B)
    def _():
        pltpu.sync_copy(al_hbm.at[wid], al_v)
        pltpu.sync_copy(ar_hbm.at[wid], ar_v)

        iota = lax.iota(jnp.int32, 16)
        w0s = [
            0.25 + 0.5 * (((iota + 16 * c) & 1).astype(jnp.float32))
            for c in range(3)
        ]

        def wpass(k, _):
            for c in range(3):
                sl = pl.ds(k * 48 + 16 * c, 16)
                aw_v[sl] = w0s[c] * al_v[sl] + (1.0 - w0s[c]) * ar_v[sl]
            return 0

        lax.fori_loop(0, 24, wpass, 0)

        for h in range(48):
            j = h // 2
            if h % 2 == 0:
                k0, wh0, k1, wh1 = max(j - 1, 0), 0.25, j, 0.75
            else:
                k0, wh0, k1, wh1 = j, 0.75, min(j + 1, 23), 0.25
            for c in range(3):
                u_v[pl.ds(h * 48 + 16 * c, 16)] = (
                    wh0 * aw_v[pl.ds(k0 * 48 + 16 * c, 16)]
                    + wh1 * aw_v[pl.ds(k1 * 48 + 16 * c, 16)]
                )

        def mmbody(i, c):
            mn, mx = c
            for q in range(6):
                uc = u_v[pl.ds(i * 96 + q * 16, 16)]
                mn = jnp.minimum(mn, uc)
                mx = jnp.maximum(mx, uc)
            return mn, mx

        mnv, mxv = lax.fori_loop(
            0, 24, mmbody, (u_v[pl.ds(0, 16)], u_v[pl.ds(0, 16)])
        )
        lo = mnv[0]
        hi = mxv[0]
        for j in range(1, 16):
            lo = jnp.minimum(lo, mnv[j])
            hi = jnp.maximum(hi, mxv[j])

        rankf = jnp.float32(rank)

        def bis(_, c):
            lo, hi = c
            mid = 0.5 * (lo + hi)

            def cnt(i, acc):
                for q in range(6):
                    uc = u_v[pl.ds(i * 96 + q * 16, 16)]
                    acc = acc + jnp.where(uc < mid, 1.0, 0.0)
                return acc

            acc = lax.fori_loop(0, 24, cnt, jnp.zeros((16,), jnp.float32))
            s = acc[0]
            for j in range(1, 16):
                s = s + acc[j]
            ok = s <= rankf
            return jnp.where(ok, mid, lo), jnp.where(ok, hi, mid)

        lo, hi = lax.fori_loop(0, _BISECT_ITERS, bis, (lo, hi))
        thr = lo

        def sbody(i, _):
            for q in range(6):
                sl = pl.ds(i * 96 + q * 16, 16)
                m_v[sl] = 1.0 / (1.0 + jnp.exp(-_TEMPERATURE * (u_v[sl] - thr)))
            return 0

        lax.fori_loop(0, 24, sbody, 0)
        pltpu.sync_copy(m_v, out_hbm.at[wid])


def _sc_mask(al, ar, rank):
    B, _ = al.shape
    mesh = plsc.VectorSubcoreMesh(core_axis_name="c", subcore_axis_name="s")
    return pl.kernel(
        functools.partial(_sc_mask_kernel, rank, B),
        out_type=jax.ShapeDtypeStruct((B, 2304), jnp.float32),
        mesh=mesh,
        scratch_types=[
            pltpu.VMEM((1152,), jnp.float32),
            pltpu.VMEM((1152,), jnp.float32),
            pltpu.VMEM((1152,), jnp.float32),
            pltpu.VMEM((2304,), jnp.float32),
            pltpu.VMEM((2304,), jnp.float32),
        ],
    )(al, ar)


def _mul_kernel(H, mt_ref, f_ref, o_ref):
    for h in range(H):
        col = mt_ref[0, :, h : h + 1] + _MIN_WEIGHT
        o_ref[0, h] = f_ref[0, h] * col


def kernel(local_feat, attention_map):
    B, C, H, W = local_feat.shape
    Bb, C1, Hg, Wg = attention_map.shape
    Hu = int(Hg * _SPATIAL_SCALE)
    Wu = int(Wg * _SPATIAL_SCALE)
    assert (Hu, Wu) == (H, W) and C1 == 1 and Bb == B
    N = Hu * Wu
    k = int(_TOPK_RATIO * N)
    rank = N - k - 1

    wv = np.arange(Wu)
    l0 = np.maximum((wv - 1) >> 1, 0)
    l1 = np.minimum((wv + 1) >> 1, Wg - 1)
    att3 = attention_map.reshape(B, Hg, Wg)
    al = att3[:, :, l0].reshape(B, Hg * Wu)
    ar = att3[:, :, l1].reshape(B, Hg * Wu)

    mask_flat = _sc_mask(al, ar, rank)
    mask = mask_flat.reshape(B, 1, Hu, Wu)
    mask_t = mask_flat.reshape(B, Hu, Wu).transpose(0, 2, 1)

    feat_t = jnp.transpose(local_feat, (0, 2, 3, 1))
    weighted_t = pl.pallas_call(
        functools.partial(_mul_kernel, H),
        grid=(B,),
        in_specs=[
            pl.BlockSpec((1, W, H), lambda b: (b, 0, 0)),
            pl.BlockSpec((1, H, W, C), lambda b: (b, 0, 0, 0)),
        ],
        out_specs=pl.BlockSpec((1, H, W, C), lambda b: (b, 0, 0, 0)),
        out_shape=jax.ShapeDtypeStruct((B, H, W, C), jnp.float32),
    )(mask_t, feat_t)
    weighted = jnp.transpose(weighted_t, (0, 3, 1, 2))

    return weighted, mask


# speedup vs baseline: 1.1824x; 1.0111x over previous
import functools

import numpy as np
import jax
import jax.numpy as jnp
from jax import lax
from jax.experimental import pallas as pl
from jax.experimental.pallas import tpu as pltpu
from jax.experimental.pallas import tpu_sc as plsc

_TOPK_RATIO = 0.3
_TEMPERATURE = 10.0
_MIN_WEIGHT = 0.1
_SPATIAL_SCALE = 2.0


def _upsample_1d_matrix(n_in: int, n_out: int) -> np.ndarray:
    U = np.zeros((n_out, n_in), np.float64)
    for i in range(n_out):
        src = (i + 0.5) * (n_in / n_out) - 0.5
        j0 = int(np.floor(src))
        f = src - j0
        j0c = min(max(j0, 0), n_in - 1)
        j1c = min(max(j0 + 1, 0), n_in - 1)
        U[i, j0c] += 1.0 - f
        U[i, j1c] += f
    return U


@functools.lru_cache(maxsize=None)
def _upsample_2d_matrix(h_in: int, w_in: int, h_out: int, w_out: int):
    UH = _upsample_1d_matrix(h_in, h_out)
    UW = _upsample_1d_matrix(w_in, w_out)
    M = np.einsum("hk,wl->klhw", UH, UW).reshape(h_in * w_in, h_out * w_out)
    return np.asarray(M, np.float32)


def _mask_kernel(rank, a_ref, m_ref, mask_ref):
    a = a_ref[...]
    u = jnp.dot(a, m_ref[...], preferred_element_type=jnp.float32)
    ibits = jax.lax.bitcast_convert_type(u, jnp.int32)
    key = ibits ^ ((ibits >> 31) & jnp.int32(0x7FFFFFFF))

    B = a.shape[0]

    def body(b, t):
        t_try = t + (jnp.int32(1) << (jnp.int32(31) - b))
        cnt = jnp.sum((key < t_try).astype(jnp.int32), axis=1, keepdims=True)
        return jnp.where(cnt <= rank, t_try, t)

    t0 = jnp.full((B, 1), jnp.int32(-2147483648))
    t = jax.lax.fori_loop(0, 32, body, t0)
    thr_i = t ^ ((t >> 31) & jnp.int32(0x7FFFFFFF))
    thr = jax.lax.bitcast_convert_type(thr_i, jnp.float32)
    mask_ref[...] = jax.nn.sigmoid(_TEMPERATURE * (u - thr))


_BISECT_ITERS = 18


def _sc_mask_kernel(rank, B, al_hbm, ar_hbm, out_hbm, al_v, ar_v, aw_v, u_v, m_v):
    nc = 2
    wid = lax.axis_index("s") * nc + lax.axis_index("c")

    @pl.when(wid < B)
    def _():
        pltpu.sync_copy(al_hbm.at[wid], al_v)
        pltpu.sync_copy(ar_hbm.at[wid], ar_v)

        iota = lax.iota(jnp.int32, 16)
        w0s = [
            0.25 + 0.5 * (((iota + 16 * c) & 1).astype(jnp.float32))
            for c in range(3)
        ]

        def wpass(k, _):
            for c in range(3):
                sl = pl.ds(k * 48 + 16 * c, 16)
                aw_v[sl] = w0s[c] * al_v[sl] + (1.0 - w0s[c]) * ar_v[sl]
            return 0

        lax.fori_loop(0, 24, wpass, 0)

        for h in range(48):
            j = h // 2
            if h % 2 == 0:
                k0, wh0, k1, wh1 = max(j - 1, 0), 0.25, j, 0.75
            else:
                k0, wh0, k1, wh1 = j, 0.75, min(j + 1, 23), 0.25
            for c in range(3):
                u_v[pl.ds(h * 48 + 16 * c, 16)] = (
                    wh0 * aw_v[pl.ds(k0 * 48 + 16 * c, 16)]
                    + wh1 * aw_v[pl.ds(k1 * 48 + 16 * c, 16)]
                )

        def mmbody(i, c):
            mn, mx = c
            for q in range(6):
                uc = u_v[pl.ds(i * 96 + q * 16, 16)]
                mn = jnp.minimum(mn, uc)
                mx = jnp.maximum(mx, uc)
            return mn, mx

        mnv, mxv = lax.fori_loop(
            0, 24, mmbody, (u_v[pl.ds(0, 16)], u_v[pl.ds(0, 16)])
        )
        lo = mnv[0]
        hi = mxv[0]
        for j in range(1, 16):
            lo = jnp.minimum(lo, mnv[j])
            hi = jnp.maximum(hi, mxv[j])

        rankf = jnp.float32(rank)

        def bis(_, c):
            lo, hi = c
            mid = 0.5 * (lo + hi)

            def cnt(i, acc):
                for q in range(12):
                    uc = u_v[pl.ds(i * 192 + q * 16, 16)]
                    acc = acc + jnp.where(uc < mid, 1.0, 0.0)
                return acc

            acc = lax.fori_loop(0, 12, cnt, jnp.zeros((16,), jnp.float32))
            s = acc[0]
            for j in range(1, 16):
                s = s + acc[j]
            ok = s <= rankf
            return jnp.where(ok, mid, lo), jnp.where(ok, hi, mid)

        lo, hi = lax.fori_loop(0, _BISECT_ITERS, bis, (lo, hi))
        thr = lo

        def sbody(i, _):
            for q in range(6):
                sl = pl.ds(i * 96 + q * 16, 16)
                m_v[sl] = 1.0 / (1.0 + jnp.exp(-_TEMPERATURE * (u_v[sl] - thr)))
            return 0

        lax.fori_loop(0, 24, sbody, 0)
        pltpu.sync_copy(m_v, out_hbm.at[wid])


def _sc_mask(al, ar, rank):
    B, _ = al.shape
    mesh = plsc.VectorSubcoreMesh(core_axis_name="c", subcore_axis_name="s")
    return pl.kernel(
        functools.partial(_sc_mask_kernel, rank, B),
        out_type=jax.ShapeDtypeStruct((B, 2304), jnp.float32),
        mesh=mesh,
        scratch_types=[
            pltpu.VMEM((1152,), jnp.float32),
            pltpu.VMEM((1152,), jnp.float32),
            pltpu.VMEM((1152,), jnp.float32),
            pltpu.VMEM((2304,), jnp.float32),
            pltpu.VMEM((2304,), jnp.float32),
        ],
    )(al, ar)


def _mul_kernel(H, mt_ref, f_ref, o_ref):
    for h in range(H):
        col = mt_ref[0, :, h : h + 1] + _MIN_WEIGHT
        o_ref[0, h] = f_ref[0, h] * col


def kernel(local_feat, attention_map):
    B, C, H, W = local_feat.shape
    Bb, C1, Hg, Wg = attention_map.shape
    Hu = int(Hg * _SPATIAL_SCALE)
    Wu = int(Wg * _SPATIAL_SCALE)
    assert (Hu, Wu) == (H, W) and C1 == 1 and Bb == B
    N = Hu * Wu
    k = int(_TOPK_RATIO * N)
    rank = N - k - 1

    wv = np.arange(Wu)
    l0 = np.maximum((wv - 1) >> 1, 0)
    l1 = np.minimum((wv + 1) >> 1, Wg - 1)
    att3 = attention_map.reshape(B, Hg, Wg)
    al = att3[:, :, l0].reshape(B, Hg * Wu)
    ar = att3[:, :, l1].reshape(B, Hg * Wu)

    mask_flat = _sc_mask(al, ar, rank)
    mask = mask_flat.reshape(B, 1, Hu, Wu)
    mask_t = mask_flat.reshape(B, Hu, Wu).transpose(0, 2, 1)

    feat_t = jnp.transpose(local_feat, (0, 2, 3, 1))
    weighted_t = pl.pallas_call(
        functools.partial(_mul_kernel, H),
        grid=(B,),
        in_specs=[
            pl.BlockSpec((1, W, H), lambda b: (b, 0, 0)),
            pl.BlockSpec((1, H, W, C), lambda b: (b, 0, 0, 0)),
        ],
        out_specs=pl.BlockSpec((1, H, W, C), lambda b: (b, 0, 0, 0)),
        out_shape=jax.ShapeDtypeStruct((B, H, W, C), jnp.float32),
    )(mask_t, feat_t)
    weighted = jnp.transpose(weighted_t, (0, 3, 1, 2))

    return weighted, mask
